# baseline (speedup 1.0000x reference)
import math

import jax
import jax.numpy as jnp
import numpy as np
from jax.experimental import pallas as pl
from jax.experimental.pallas import tpu as pltpu

D = 128
H = 8
FF = 512
DEPTH = 3
E = 8
HID = 256
EPS = 0.1
TILE = 8192

_LOG_E = np.float32(np.log(E + 1e-9))
_INV_SQRT2 = np.float32(1.0 / math.sqrt(2.0))


def _ln(h, g, b):
    m = jnp.mean(h, axis=-1, keepdims=True)
    v = jnp.mean((h - m) ** 2, axis=-1, keepdims=True)
    return (h - m) / jnp.sqrt(v + 1e-5) * g + b


def _gelu2(u):
    return u + u * jax.lax.erf(u * _INV_SQRT2)


def _dot(a, b):
    return jnp.dot(a, b, preferred_element_type=jnp.float32)


def _fwd_kernel(x_ref, eWt, eb, l1g, l1b, Wvt, bv, l2g, l2b,
                W1t, b1, W2t, b2, rWt, rb, xW1t, xb1, xW2t, xb2,
                hg, hb, hWt, hb0,
                logit_ref, aux_ref, psum_ref):
    i = pl.program_id(0)
    nb = pl.num_programs(0)
    h = _dot(x_ref[...], eWt[...]) + eb[...]
    for d in range(DEPTH):
        hn = _ln(h, l1g[d], l1b[d])
        h = h + _dot(hn, Wvt[d]) + bv[d]
        hn = _ln(h, l2g[d], l2b[d])
        ff = _gelu2(_dot(hn, W1t[d]) + b1[d])
        h = h + _dot(ff, W2t[d]) + b2[d]
    z = h

    logits = _dot(z, rWt[...]) + rb[...]
    mx = jnp.max(logits, axis=-1, keepdims=True)
    ex = jnp.exp(logits - mx)
    probs = ex / jnp.sum(ex, axis=-1, keepdims=True)

    iota = jax.lax.broadcasted_iota(jnp.int32, probs.shape, 1)
    m1 = jnp.max(probs, axis=-1, keepdims=True)
    i1 = jnp.min(jnp.where(probs == m1, iota, E), axis=-1, keepdims=True)
    pm = jnp.where(iota == i1, -jnp.inf, probs)
    m2 = jnp.max(pm, axis=-1, keepdims=True)
    i2 = jnp.min(jnp.where(pm == m2, iota, E), axis=-1, keepdims=True)
    w1 = (1.0 - EPS) * m1 + (EPS / E)
    w2 = (1.0 - EPS) * m2 + (EPS / E)
    sel = (jnp.where(iota == i1, w1, 0.0) + jnp.where(iota == i2, w2, 0.0))

    zb = z.astype(jnp.bfloat16)
    acc = jnp.zeros_like(z)
    for e_ in range(E):
        he = _gelu2(_dot(zb, xW1t[e_]) + xb1[e_])
        oe = _dot(he.astype(jnp.bfloat16), xW2t[e_]) + xb2[e_]
        acc = acc + sel[:, e_:e_ + 1] * oe
    z = z + acc

    zn = _ln(z, hg[...], hb[...])
    logit_ref[...] = _dot(zn, hWt[...]) + hb0[...]

    ps = jnp.sum(probs - np.float32(1.0 / E), axis=0, keepdims=True)

    @pl.when(i == 0)
    def _():
        psum_ref[...] = ps

    @pl.when(i > 0)
    def _():
        psum_ref[...] = psum_ref[...] + ps

    @pl.when(i == nb - 1)
    def _():
        inv_n = np.float32(1.0 - EPS) / (jnp.float32(nb) * TILE)
        load = psum_ref[...] * inv_n + np.float32(1.0 / E)
        aux_ref[...] = jnp.sum(
            load * jnp.log(load * E + 1e-9), axis=-1, keepdims=True) / _LOG_E


def kernel(x, params):
    p = params
    n_tok, n_in = x.shape
    nb = n_tok // TILE

    blks = p['blocks']
    eWt = p['embed_W'].T
    eb = p['embed_b'][None]
    l1g = jnp.stack([b['ln1_g'] for b in blks])[:, None, :]
    l1b = jnp.stack([b['ln1_b'] for b in blks])[:, None, :]
    Wvt = jnp.stack(
        [jnp.dot(b['Wqkv'][2 * D:].T, b['Wo'].T,
                 preferred_element_type=jnp.float32) for b in blks])
    bv = jnp.stack(
        [jnp.dot(b['bqkv'][2 * D:], b['Wo'].T,
                 preferred_element_type=jnp.float32) + b['bo']
         for b in blks])[:, None, :]
    l2g = jnp.stack([b['ln2_g'] for b in blks])[:, None, :]
    l2b = jnp.stack([b['ln2_b'] for b in blks])[:, None, :]
    W1t = jnp.stack([b['W1'].T for b in blks])
    b1 = jnp.stack([b['b1'] for b in blks])[:, None, :]
    W2t = 0.5 * jnp.stack([b['W2'].T for b in blks])
    b2 = jnp.stack([b['b2'] for b in blks])[:, None, :]
    rWt = p['router_W'].T
    rb = p['router_b'][None]
    xW1t = p['exp_W1'].transpose(0, 2, 1).astype(jnp.bfloat16)
    xb1 = p['exp_b1'][:, None, :]
    xW2t = (0.5 * p['exp_W2'].transpose(0, 2, 1)).astype(jnp.bfloat16)
    xb2 = p['exp_b2'][:, None, :]
    hg = p['head_ln_g'][None]
    hb = p['head_ln_b'][None]
    hWt = p['head_W'].T
    hb0 = p['head_b'][None]

    def full(a):
        return pl.BlockSpec(a.shape, lambda i: (0,) * a.ndim)

    args = (eWt, eb, l1g, l1b, Wvt, bv, l2g, l2b,
            W1t, b1, W2t, b2, rWt, rb, xW1t, xb1, xW2t, xb2,
            hg, hb, hWt, hb0)

    logit, aux = pl.pallas_call(
        _fwd_kernel,
        grid=(nb,),
        in_specs=[pl.BlockSpec((TILE, n_in), lambda i: (i, 0))]
                 + [full(a) for a in args],
        out_specs=[pl.BlockSpec((TILE, 1), lambda i: (i, 0)),
                   pl.BlockSpec((1, 1), lambda i: (0, 0))],
        out_shape=[jax.ShapeDtypeStruct((n_tok, 1), jnp.float32),
                   jax.ShapeDtypeStruct((1, 1), jnp.float32)],
        scratch_shapes=[pltpu.VMEM((1, E), jnp.float32)],
    )(x, *args)
    return logit[:, 0], aux[0, 0]


# speedup vs baseline: 1.2390x; 1.2390x over previous
import math

import jax
import jax.numpy as jnp
import numpy as np
from jax.experimental import pallas as pl
from jax.experimental.pallas import tpu as pltpu

D = 128
H = 8
FF = 512
DEPTH = 3
E = 8
HID = 256
EPS = 0.1
TILE = 4096

_LOG_E = np.float32(np.log(E + 1e-9))
_INV_SQRT2 = np.float32(1.0 / math.sqrt(2.0))


def _ln(h, g, b):
    m = jnp.mean(h, axis=-1, keepdims=True)
    v = jnp.mean((h - m) ** 2, axis=-1, keepdims=True)
    return (h - m) / jnp.sqrt(v + 1e-5) * g + b


def _gelu2(u):
    return u + u * jax.lax.erf(u * _INV_SQRT2)


def _dot(a, b):
    return jnp.dot(a, b, preferred_element_type=jnp.float32)


def _fwd_kernel(x_ref, eWt, eb, l1g, l1b, Wvt, bv, l2g, l2b,
                W1t, b1, W2t, b2, rWt, rb, xW1t, xb1, xW2t, xb2,
                hg, hb, hWt, hb0,
                logit_ref, aux_ref, psum_ref):
    i = pl.program_id(0)
    nb = pl.num_programs(0)
    h = _dot(x_ref[...], eWt[...]) + eb[...]
    for d in range(DEPTH):
        hn = _ln(h, l1g[d], l1b[d])
        h = h + _dot(hn, Wvt[d]) + bv[d]
        hn = _ln(h, l2g[d], l2b[d])
        ff = _gelu2(_dot(hn, W1t[d]) + b1[d])
        h = h + _dot(ff, W2t[d]) + b2[d]
    z = h

    logits = _dot(z, rWt[...]) + rb[...]
    mx = jnp.max(logits, axis=-1, keepdims=True)
    ex = jnp.exp(logits - mx)
    probs = ex / jnp.sum(ex, axis=-1, keepdims=True)

    iota = jax.lax.broadcasted_iota(jnp.int32, probs.shape, 1)
    m1 = jnp.max(probs, axis=-1, keepdims=True)
    i1 = jnp.min(jnp.where(probs == m1, iota, E), axis=-1, keepdims=True)
    pm = jnp.where(iota == i1, -jnp.inf, probs)
    m2 = jnp.max(pm, axis=-1, keepdims=True)
    i2 = jnp.min(jnp.where(pm == m2, iota, E), axis=-1, keepdims=True)
    w1 = (1.0 - EPS) * m1 + (EPS / E)
    w2 = (1.0 - EPS) * m2 + (EPS / E)
    sel = (jnp.where(iota == i1, w1, 0.0) + jnp.where(iota == i2, w2, 0.0))

    zb = z.astype(jnp.bfloat16)
    acc = jnp.zeros_like(z)
    for e_ in range(E):
        he = _gelu2(_dot(zb, xW1t[e_]) + xb1[e_])
        oe = _dot(he.astype(jnp.bfloat16), xW2t[e_]) + xb2[e_]
        acc = acc + sel[:, e_:e_ + 1] * oe
    z = z + acc

    zn = _ln(z, hg[...], hb[...])
    logit_ref[...] = _dot(zn, hWt[...]) + hb0[...]

    ps = jnp.sum(probs - np.float32(1.0 / E), axis=0, keepdims=True)

    @pl.when(i == 0)
    def _():
        psum_ref[...] = ps

    @pl.when(i > 0)
    def _():
        psum_ref[...] = psum_ref[...] + ps

    @pl.when(i == nb - 1)
    def _():
        inv_n = np.float32(1.0 - EPS) / (jnp.float32(nb) * TILE)
        load = psum_ref[...] * inv_n + np.float32(1.0 / E)
        aux_ref[...] = jnp.sum(
            load * jnp.log(load * E + 1e-9), axis=-1, keepdims=True) / _LOG_E


def kernel(x, params):
    p = params
    n_tok, n_in = x.shape
    nb = n_tok // TILE

    blks = p['blocks']
    eWt = p['embed_W'].T
    eb = p['embed_b'][None]
    l1g = jnp.stack([b['ln1_g'] for b in blks])[:, None, :]
    l1b = jnp.stack([b['ln1_b'] for b in blks])[:, None, :]
    Wvt = jnp.stack(
        [jnp.dot(b['Wqkv'][2 * D:].T, b['Wo'].T,
                 preferred_element_type=jnp.float32) for b in blks])
    bv = jnp.stack(
        [jnp.dot(b['bqkv'][2 * D:], b['Wo'].T,
                 preferred_element_type=jnp.float32) + b['bo']
         for b in blks])[:, None, :]
    l2g = jnp.stack([b['ln2_g'] for b in blks])[:, None, :]
    l2b = jnp.stack([b['ln2_b'] for b in blks])[:, None, :]
    W1t = jnp.stack([b['W1'].T for b in blks])
    b1 = jnp.stack([b['b1'] for b in blks])[:, None, :]
    W2t = 0.5 * jnp.stack([b['W2'].T for b in blks])
    b2 = jnp.stack([b['b2'] for b in blks])[:, None, :]
    rWt = p['router_W'].T
    rb = p['router_b'][None]
    xW1t = p['exp_W1'].transpose(0, 2, 1).astype(jnp.bfloat16)
    xb1 = p['exp_b1'][:, None, :]
    xW2t = (0.5 * p['exp_W2'].transpose(0, 2, 1)).astype(jnp.bfloat16)
    xb2 = p['exp_b2'][:, None, :]
    hg = p['head_ln_g'][None]
    hb = p['head_ln_b'][None]
    hWt = p['head_W'].T
    hb0 = p['head_b'][None]

    def full(a):
        return pl.BlockSpec(a.shape, lambda i: (0,) * a.ndim)

    args = (eWt, eb, l1g, l1b, Wvt, bv, l2g, l2b,
            W1t, b1, W2t, b2, rWt, rb, xW1t, xb1, xW2t, xb2,
            hg, hb, hWt, hb0)

    logit, aux = pl.pallas_call(
        _fwd_kernel,
        grid=(nb,),
        in_specs=[pl.BlockSpec((TILE, n_in), lambda i: (i, 0))]
                 + [full(a) for a in args],
        out_specs=[pl.BlockSpec((TILE, 1), lambda i: (i, 0)),
                   pl.BlockSpec((1, 1), lambda i: (0, 0))],
        out_shape=[jax.ShapeDtypeStruct((n_tok, 1), jnp.float32),
                   jax.ShapeDtypeStruct((1, 1), jnp.float32)],
        scratch_shapes=[pltpu.VMEM((1, E), jnp.float32)],
    )(x, *args)
    return logit[:, 0], aux[0, 0]
